# baseline (speedup 1.0000x reference)
import functools
import math

import numpy as np
import jax
import jax.numpy as jnp
from jax import lax
from jax.experimental import pallas as pl
from jax.experimental.pallas import tpu as pltpu
from jax.experimental.pallas import tpu_sc as plsc

DIM = 768
HEADS = 12
HD = 64
KN = 64
SEQ = 2048
SCALE = 1.0 / math.sqrt(HD)

NW = 32
CQ = SEQ // NW
UROW = 128
NUG = 2
UP = NUG * UROW


def _cantor_coords_np(seq_len, depth=8):
    pos = np.arange(seq_len)
    x = (pos.astype(np.float32) / np.float32(max(1, seq_len - 1))).astype(np.float32)
    x = np.clip(x, np.float32(1e-06), np.float32(1.0 - 1e-06)).astype(np.float32)
    val = np.zeros_like(x)
    factor = 0.5
    for _ in range(depth):
        xs = (x * np.float32(3.0)).astype(np.float32)
        digit = xs.astype(np.int32)
        xf = (xs - digit.astype(np.float32)).astype(np.float32)
        val = (val + (digit == 2).astype(np.float32) * np.float32(factor)).astype(np.float32)
        x = xf
        factor *= 0.5
    return np.clip(val, 0.0, 1.0).astype(np.float32)


@functools.lru_cache(maxsize=1)
def _route_structure():
    coords = _cantor_coords_np(SEQ)
    dist = np.abs(coords[:, None] - coords[None, :])
    routes = np.argsort(dist, axis=1, kind="stable")[:, :KN].astype(np.int32)
    order = np.argsort(coords, kind="stable").astype(np.int32)
    uidx = np.zeros((NW, UP), dtype=np.int32)
    lidx = np.zeros((NW, CQ * KN), dtype=np.int32)
    qpos = np.zeros((NW, 1, CQ), dtype=np.int32)
    for c in range(NW):
        qs = order[c * CQ:(c + 1) * CQ]
        sub = routes[qs]
        uniq = np.unique(sub)
        if len(uniq) > UP:
            raise ValueError("route union exceeds padded capacity")
        uidx[c, :len(uniq)] = uniq
        lidx[c] = np.searchsorted(uniq, sub.ravel()).astype(np.int32)
        qpos[c, 0] = qs
    return (uidx.reshape(NW, NUG, UROW), lidx, qpos)


def _mm_bias_body(x_ref, w_ref, b_ref, o_ref):
    acc = lax.dot_general(x_ref[...], w_ref[...], (((1,), (1,)), ((), ())),
                          preferred_element_type=jnp.float32)
    o_ref[...] = acc + b_ref[...]


def _matmul_bias(x2d, w, b2d, bm=256, bn=256):
    m, kd = x2d.shape
    n = w.shape[0]
    return pl.pallas_call(
        _mm_bias_body,
        grid=(m // bm, n // bn),
        in_specs=[
            pl.BlockSpec((bm, kd), lambda i, j: (i, 0)),
            pl.BlockSpec((bn, kd), lambda i, j: (j, 0)),
            pl.BlockSpec((1, bn), lambda i, j: (0, j)),
        ],
        out_specs=pl.BlockSpec((bm, bn), lambda i, j: (i, j)),
        out_shape=jax.ShapeDtypeStruct((m, n), jnp.float32),
    )(x2d, w, b2d)


_TAKE_DNUMS = lax.GatherDimensionNumbers(
    offset_dims=(), collapsed_slice_dims=(0,), start_index_map=(0,))


def _lane_take(vec, lane):
    return lax.gather(vec, lane[:, None], _TAKE_DNUMS, (1,),
                      mode=lax.GatherScatterMode.PROMISE_IN_BOUNDS)


def _sc_attention(qkv_r, uidx, lidx, qpos):
    mesh = plsc.VectorSubcoreMesh(core_axis_name="c", subcore_axis_name="s")

    @functools.partial(
        pl.kernel,
        out_type=jax.ShapeDtypeStruct((HEADS, SEQ, HD), jnp.float32),
        mesh=mesh,
        compiler_params=pltpu.CompilerParams(needs_layout_passes=False,
                                             use_tc_tiling_on_sc=False),
        scratch_types=[
            pltpu.VMEM((NUG, UROW), jnp.int32),
            pltpu.VMEM((CQ * KN,), jnp.int32),
            pltpu.VMEM((1, CQ), jnp.int32),
            pltpu.VMEM((CQ, HD), jnp.float32),
            pltpu.VMEM((UP, HD), jnp.float32),
            pltpu.VMEM((UP, HD), jnp.float32),
            pltpu.VMEM((CQ, HD), jnp.float32),
            pltpu.VMEM((UP, HD), jnp.float32),
            pltpu.VMEM((UP, HD), jnp.float32),
            pltpu.VMEM((CQ, HD), jnp.float32),
            pltpu.SemaphoreType.DMA,
            pltpu.SemaphoreType.DMA,
        ],
    )
    def body(qkv_hbm, uidx_hbm, lidx_hbm, qpos_hbm, out_hbm,
             uidx_v, lidx_v, qpos_v, q_v0, k_v0, v_v0, q_v1, k_v1, v_v1,
             o_v, sem0, sem1):
        wid = lax.axis_index("c") * 16 + lax.axis_index("s")
        pltpu.sync_copy(uidx_hbm.at[wid], uidx_v)
        pltpu.sync_copy(lidx_hbm.at[wid], lidx_v)
        pltpu.sync_copy(qpos_hbm.at[wid], qpos_v)

        iota = lax.iota(jnp.int32, 16)
        cols = [iota + g * 16 for g in range(4)]
        zero4 = tuple(jnp.zeros((16,), jnp.float32) for _ in range(4))

        def descs(h, q_v, k_v, v_v, sem):
            ds = [pltpu.make_async_copy(qkv_hbm.at[h].at[qpos_v.at[0]], q_v, sem)]
            for ug in range(NUG):
                ds.append(
                    pltpu.make_async_copy(qkv_hbm.at[HEADS + h].at[uidx_v.at[ug]],
                                          k_v.at[pl.ds(ug * UROW, UROW)], sem))
                ds.append(
                    pltpu.make_async_copy(qkv_hbm.at[2 * HEADS + h].at[uidx_v.at[ug]],
                                          v_v.at[pl.ds(ug * UROW, UROW)], sem))
            return ds

        def issue(h, q_v, k_v, v_v, sem):
            for d in descs(h, q_v, k_v, v_v, sem):
                d.start()

        def drain(h, q_v, k_v, v_v, sem):
            for d in descs(h, q_v, k_v, v_v, sem):
                d.wait()

        def compute_head(h, q_v, k_v, v_v):
            def one_query(qloc):
                qsplat = jnp.full((16,), qloc, jnp.int32)
                lrow = [lidx_v[pl.ds(qloc * KN + g * 16, 16)] for g in range(4)]
                qv = [plsc.load_gather(q_v, [qsplat, cols[g]]) for g in range(4)]

                svecs = []
                for ng in range(4):
                    sacc = jnp.zeros((16,), jnp.float32)
                    for i in range(16):
                        row = lrow[ng][i]
                        p = qv[0] * k_v[row, pl.ds(0, 16)]
                        for g in range(1, 4):
                            p = p + qv[g] * k_v[row, pl.ds(g * 16, 16)]
                        sacc = jnp.where(iota == i,
                                         jnp.full((16,), jnp.sum(p), jnp.float32),
                                         sacc)
                    svecs.append(sacc * jnp.float32(SCALE))

                m = jnp.max(jnp.maximum(jnp.maximum(svecs[0], svecs[1]),
                                        jnp.maximum(svecs[2], svecs[3])))
                e = [jnp.exp(a - jnp.full((16,), m, jnp.float32)) for a in svecs]
                tot = jnp.sum(e[0] + e[1] + e[2] + e[3])
                totv = jnp.full((16,), tot, jnp.float32)
                w = [e[g] / totv for g in range(4)]

                oaccs = list(zero4)
                for ng in range(4):
                    for i in range(16):
                        av = jnp.full((16,), w[ng][i], jnp.float32)
                        row = lrow[ng][i]
                        for g in range(4):
                            oaccs[g] = oaccs[g] + av * v_v[row, pl.ds(g * 16, 16)]
                for g in range(4):
                    plsc.store_scatter(o_v, [qsplat, cols[g]], oaccs[g])

            def q_body(qp, qcarry):
                one_query(2 * qp)
                one_query(2 * qp + 1)
                return qcarry

            lax.fori_loop(0, CQ // 2, q_body, 0)
            pltpu.sync_copy(o_v, out_hbm.at[h].at[qpos_v.at[0]])

        issue(0, q_v0, k_v0, v_v0, sem0)

        def pair_body(hp, carry):
            h0 = 2 * hp
            issue(h0 + 1, q_v1, k_v1, v_v1, sem1)
            drain(h0, q_v0, k_v0, v_v0, sem0)
            compute_head(h0, q_v0, k_v0, v_v0)

            @pl.when(h0 + 2 < HEADS)
            def _():
                issue(h0 + 2, q_v0, k_v0, v_v0, sem0)

            drain(h0 + 1, q_v1, k_v1, v_v1, sem1)
            compute_head(h0 + 1, q_v1, k_v1, v_v1)
            return carry

        lax.fori_loop(0, HEADS // 2, pair_body, 0)

    return body(qkv_r, uidx, lidx, qpos)


def kernel(x, W_qkv, b_qkv, W_out, b_out):
    b, s, d = x.shape
    uidx, lidx, qpos = _route_structure()
    x2d = x.reshape(s, d)
    qkv = _matmul_bias(x2d, W_qkv, b_qkv.reshape(1, -1))
    qkv_r = qkv.reshape(s, 3 * HEADS, HD).transpose(1, 0, 2)
    attn = _sc_attention(qkv_r, jnp.asarray(uidx), jnp.asarray(lidx),
                         jnp.asarray(qpos))
    attn2d = attn.transpose(1, 0, 2).reshape(s, d)
    out = _matmul_bias(attn2d, W_out, b_out.reshape(1, -1))
    return out.reshape(b, s, d)


# speedup vs baseline: 1.2206x; 1.2206x over previous
import functools
import math

import numpy as np
import jax
import jax.numpy as jnp
from jax import lax
from jax.experimental import pallas as pl
from jax.experimental.pallas import tpu as pltpu
from jax.experimental.pallas import tpu_sc as plsc

DIM = 768
HEADS = 12
HD = 64
KN = 64
SEQ = 2048
SCALE = 1.0 / math.sqrt(HD)

NW = 32
CQ = SEQ // NW
UROW = 128
NUG = 2
UP = NUG * UROW


def _cantor_coords_np(seq_len, depth=8):
    pos = np.arange(seq_len)
    x = (pos.astype(np.float32) / np.float32(max(1, seq_len - 1))).astype(np.float32)
    x = np.clip(x, np.float32(1e-06), np.float32(1.0 - 1e-06)).astype(np.float32)
    val = np.zeros_like(x)
    factor = 0.5
    for _ in range(depth):
        xs = (x * np.float32(3.0)).astype(np.float32)
        digit = xs.astype(np.int32)
        xf = (xs - digit.astype(np.float32)).astype(np.float32)
        val = (val + (digit == 2).astype(np.float32) * np.float32(factor)).astype(np.float32)
        x = xf
        factor *= 0.5
    return np.clip(val, 0.0, 1.0).astype(np.float32)


@functools.lru_cache(maxsize=1)
def _route_structure():
    coords = _cantor_coords_np(SEQ)
    dist = np.abs(coords[:, None] - coords[None, :])
    routes = np.argsort(dist, axis=1, kind="stable")[:, :KN].astype(np.int32)
    order = np.argsort(coords, kind="stable").astype(np.int32)
    uidx = np.zeros((NW, UP), dtype=np.int32)
    lidx = np.zeros((NW, CQ * KN), dtype=np.int32)
    qpos = np.zeros((NW, 1, CQ), dtype=np.int32)
    for c in range(NW):
        qs = order[c * CQ:(c + 1) * CQ]
        sub = routes[qs]
        uniq = np.unique(sub)
        if len(uniq) > UP:
            raise ValueError("route union exceeds padded capacity")
        uidx[c, :len(uniq)] = uniq
        lidx[c] = np.searchsorted(uniq, sub.ravel()).astype(np.int32)
        qpos[c, 0] = qs
    return (uidx.reshape(NW, NUG, UROW), lidx, qpos)


def _mm_bias_body(x_ref, w_ref, b_ref, o_ref):
    acc = lax.dot_general(x_ref[...], w_ref[...], (((1,), (1,)), ((), ())),
                          preferred_element_type=jnp.float32)
    o_ref[...] = acc + b_ref[...]


def _matmul_bias(x2d, w, b2d, bm=256, bn=256):
    m, kd = x2d.shape
    n = w.shape[0]
    return pl.pallas_call(
        _mm_bias_body,
        grid=(m // bm, n // bn),
        in_specs=[
            pl.BlockSpec((bm, kd), lambda i, j: (i, 0)),
            pl.BlockSpec((bn, kd), lambda i, j: (j, 0)),
            pl.BlockSpec((1, bn), lambda i, j: (0, j)),
        ],
        out_specs=pl.BlockSpec((bm, bn), lambda i, j: (i, j)),
        out_shape=jax.ShapeDtypeStruct((m, n), jnp.float32),
    )(x2d, w, b2d)


_TAKE_DNUMS = lax.GatherDimensionNumbers(
    offset_dims=(), collapsed_slice_dims=(0,), start_index_map=(0,))


def _lane_take(vec, lane):
    return lax.gather(vec, lane[:, None], _TAKE_DNUMS, (1,),
                      mode=lax.GatherScatterMode.PROMISE_IN_BOUNDS)


def _sc_attention(qkv_r, uidx, lidx, qpos):
    mesh = plsc.VectorSubcoreMesh(core_axis_name="c", subcore_axis_name="s")

    @functools.partial(
        pl.kernel,
        out_type=jax.ShapeDtypeStruct((HEADS, SEQ, HD), jnp.float32),
        mesh=mesh,
        compiler_params=pltpu.CompilerParams(needs_layout_passes=False,
                                             use_tc_tiling_on_sc=False),
        scratch_types=[
            pltpu.VMEM((NUG, UROW), jnp.int32),
            pltpu.VMEM((CQ * KN,), jnp.int32),
            pltpu.VMEM((1, CQ), jnp.int32),
            pltpu.VMEM((CQ, HD), jnp.bfloat16),
            pltpu.VMEM((UP, HD), jnp.bfloat16),
            pltpu.VMEM((UP, HD), jnp.bfloat16),
            pltpu.VMEM((CQ, HD), jnp.bfloat16),
            pltpu.VMEM((UP, HD), jnp.bfloat16),
            pltpu.VMEM((UP, HD), jnp.bfloat16),
            pltpu.VMEM((CQ, HD), jnp.float32),
            pltpu.SemaphoreType.DMA,
            pltpu.SemaphoreType.DMA,
        ],
    )
    def body(qkv_hbm, uidx_hbm, lidx_hbm, qpos_hbm, out_hbm,
             uidx_v, lidx_v, qpos_v, q_v0, k_v0, v_v0, q_v1, k_v1, v_v1,
             o_v, sem0, sem1):
        wid = lax.axis_index("c") * 16 + lax.axis_index("s")
        pltpu.sync_copy(uidx_hbm.at[wid], uidx_v)
        pltpu.sync_copy(lidx_hbm.at[wid], lidx_v)
        pltpu.sync_copy(qpos_hbm.at[wid], qpos_v)

        iota = lax.iota(jnp.int32, 16)
        cols = [iota + g * 16 for g in range(4)]
        ocols = [iota * 2, iota * 2 + 1, iota * 2 + 32, iota * 2 + 33]
        zero4 = tuple(jnp.zeros((16,), jnp.float32) for _ in range(4))

        def load_row_f32(ref, row):
            h0 = ref[row, pl.ds(0, 32)]
            h1 = ref[row, pl.ds(32, 32)]
            a0, b0 = plsc.unpack(h0, format=plsc.PackFormat.INTERLEAVED)
            a1, b1 = plsc.unpack(h1, format=plsc.PackFormat.INTERLEAVED)
            return (a0, b0, a1, b1)

        def descs(h, q_v, k_v, v_v, sem):
            ds = [pltpu.make_async_copy(qkv_hbm.at[h].at[qpos_v.at[0]], q_v, sem)]
            for ug in range(NUG):
                ds.append(
                    pltpu.make_async_copy(qkv_hbm.at[HEADS + h].at[uidx_v.at[ug]],
                                          k_v.at[pl.ds(ug * UROW, UROW)], sem))
                ds.append(
                    pltpu.make_async_copy(qkv_hbm.at[2 * HEADS + h].at[uidx_v.at[ug]],
                                          v_v.at[pl.ds(ug * UROW, UROW)], sem))
            return ds

        def issue(h, q_v, k_v, v_v, sem):
            for d in descs(h, q_v, k_v, v_v, sem):
                d.start()

        def drain(h, q_v, k_v, v_v, sem):
            for d in descs(h, q_v, k_v, v_v, sem):
                d.wait()

        def compute_head(h, q_v, k_v, v_v):
            def one_query(qloc):
                qsplat = jnp.full((16,), qloc, jnp.int32)
                lrow = [lidx_v[pl.ds(qloc * KN + g * 16, 16)] for g in range(4)]
                qv = load_row_f32(q_v, qloc)

                svecs = []
                for ng in range(4):
                    sacc = jnp.zeros((16,), jnp.float32)
                    for i in range(16):
                        kk = load_row_f32(k_v, lrow[ng][i])
                        p = qv[0] * kk[0]
                        for g in range(1, 4):
                            p = p + qv[g] * kk[g]
                        sacc = jnp.where(iota == i,
                                         jnp.full((16,), jnp.sum(p), jnp.float32),
                                         sacc)
                    svecs.append(sacc * jnp.float32(SCALE))

                m = jnp.max(jnp.maximum(jnp.maximum(svecs[0], svecs[1]),
                                        jnp.maximum(svecs[2], svecs[3])))
                e = [jnp.exp(a - jnp.full((16,), m, jnp.float32)) for a in svecs]
                tot = jnp.sum(e[0] + e[1] + e[2] + e[3])
                totv = jnp.full((16,), tot, jnp.float32)
                w = [e[g] / totv for g in range(4)]

                oaccs = list(zero4)
                for ng in range(4):
                    for i in range(16):
                        av = jnp.full((16,), w[ng][i], jnp.float32)
                        vv = load_row_f32(v_v, lrow[ng][i])
                        for g in range(4):
                            oaccs[g] = oaccs[g] + av * vv[g]
                for g in range(4):
                    plsc.store_scatter(o_v, [qsplat, ocols[g]], oaccs[g])

            def q_body(qloc, qcarry):
                one_query(qloc)
                return qcarry

            lax.fori_loop(0, CQ, q_body, 0)
            pltpu.sync_copy(o_v, out_hbm.at[h].at[qpos_v.at[0]])

        issue(0, q_v0, k_v0, v_v0, sem0)

        def pair_body(hp, carry):
            h0 = 2 * hp
            issue(h0 + 1, q_v1, k_v1, v_v1, sem1)
            drain(h0, q_v0, k_v0, v_v0, sem0)
            compute_head(h0, q_v0, k_v0, v_v0)

            @pl.when(h0 + 2 < HEADS)
            def _():
                issue(h0 + 2, q_v0, k_v0, v_v0, sem0)

            drain(h0 + 1, q_v1, k_v1, v_v1, sem1)
            compute_head(h0 + 1, q_v1, k_v1, v_v1)
            return carry

        lax.fori_loop(0, HEADS // 2, pair_body, 0)

    return body(qkv_r, uidx, lidx, qpos)


def kernel(x, W_qkv, b_qkv, W_out, b_out):
    b, s, d = x.shape
    uidx, lidx, qpos = _route_structure()
    x2d = x.reshape(s, d)
    qkv = _matmul_bias(x2d, W_qkv, b_qkv.reshape(1, -1))
    qkv_r = qkv.reshape(s, 3 * HEADS, HD).transpose(1, 0, 2)
    attn = _sc_attention(qkv_r.astype(jnp.bfloat16),
                         jnp.asarray(uidx), jnp.asarray(lidx),
                         jnp.asarray(qpos))
    attn2d = attn.transpose(1, 0, 2).reshape(s, d)
    out = _matmul_bias(attn2d, W_out, b_out.reshape(1, -1))
    return out.reshape(b, s, d)
